# Initial kernel scaffold; baseline (speedup 1.0000x reference)
import jax
import jax.numpy as jnp
from jax.experimental import pallas as pl


def kernel(dense_features, categorical_ids, field_offsets, table, W, b):
    raise NotImplementedError("write your pallas kernel here")


# speedup vs baseline: 16.2394x; 16.2394x over previous
import functools

import jax
import jax.numpy as jnp
from jax import lax
from jax.experimental import pallas as pl
from jax.experimental.pallas import tpu as pltpu
from jax.experimental.pallas import tpu_sc as plsc

B = 16384
F = 26
EMB = 16
ND = 8
OUT = 64

NC = 2
NS = 16
NW = NC * NS
LANES = 16

N_FLAT = B * F
N_PER_W = N_FLAT // NW
PERIOD = 208
N_CHUNK = 16
G = N_PER_W // N_CHUNK


def _sc_gather_body(ids_hbm, pat_hbm, table_hbm, out_hbm, idx_v, pat_v, rows_v, gsem):
    wid = lax.axis_index("s") * NC + lax.axis_index("c")
    base = wid * N_PER_W
    pltpu.sync_copy(ids_hbm.at[pl.ds(base, N_PER_W)], idx_v)
    pltpu.sync_copy(pat_hbm, pat_v)

    pats = tuple(pat_v[pl.ds(j * LANES, LANES)] for j in range(PERIOD // LANES))

    def tbody(t, pats):
        p0 = t * PERIOD
        for j in range(PERIOD // LANES):
            sl = pl.ds(p0 + j * LANES, LANES)
            idx_v[sl] = idx_v[sl] + pats[j]
        return pats

    lax.fori_loop(0, N_PER_W // PERIOD, tbody, pats)

    def gbody(c, carry):
        off = c * G
        pltpu.async_copy(table_hbm.at[idx_v.at[pl.ds(off, G)]], rows_v, gsem).wait()
        pltpu.sync_copy(rows_v, out_hbm.at[pl.ds(base + off, G)])
        return carry

    lax.fori_loop(0, N_CHUNK, gbody, 0)


@jax.jit
def _sc_gather(ids_flat, pattern, table):
    mesh = plsc.VectorSubcoreMesh(core_axis_name="c", subcore_axis_name="s")
    f = pl.kernel(
        _sc_gather_body,
        out_type=jax.ShapeDtypeStruct((N_FLAT, EMB), jnp.float32),
        mesh=mesh,
        scratch_types=[
            pltpu.VMEM((N_PER_W,), jnp.int32),
            pltpu.VMEM((PERIOD,), jnp.int32),
            pltpu.VMEM((G, EMB), jnp.float32),
            pltpu.SemaphoreType.DMA,
        ],
        compiler_params=pltpu.CompilerParams(use_tc_tiling_on_sc=False),
    )
    return f(ids_flat, pattern, table)


def _mlp_body(dense_ref, emb_ref, wd_ref, wc_ref, b_ref, out_ref):
    acc = jnp.dot(dense_ref[...], wd_ref[...], preferred_element_type=jnp.float32)
    acc = acc + jnp.dot(emb_ref[...], wc_ref[...], preferred_element_type=jnp.float32)
    acc = acc + b_ref[...]
    out_ref[...] = 0.5 * acc * (1.0 + lax.erf(acc * (2.0 ** -0.5)))


@functools.partial(jax.jit, static_argnames=("bm",))
def _mlp(dense, emb, wd, wc, b, bm=2048):
    grid = (B // bm,)
    return pl.pallas_call(
        _mlp_body,
        grid=grid,
        in_specs=[
            pl.BlockSpec((bm, ND), lambda i: (i, 0)),
            pl.BlockSpec((bm, F * EMB), lambda i: (i, 0)),
            pl.BlockSpec((ND, OUT), lambda i: (0, 0)),
            pl.BlockSpec((F * EMB, OUT), lambda i: (0, 0)),
            pl.BlockSpec((1, OUT), lambda i: (0, 0)),
        ],
        out_specs=pl.BlockSpec((bm, OUT), lambda i: (i, 0)),
        out_shape=jax.ShapeDtypeStruct((B, OUT), jnp.float32),
    )(dense, emb, wd, wc, b)


def kernel(dense_features, categorical_ids, field_offsets, table, W, b):
    ids_flat = categorical_ids.reshape(-1)
    pattern = jnp.tile(field_offsets, PERIOD // F)
    emb = _sc_gather(ids_flat, pattern, table)
    emb2d = emb.reshape(B, F * EMB)
    wd = W[:ND]
    wc = W[ND:]
    return _mlp(dense_features, emb2d, wd, wc, b.reshape(1, OUT))
